# Initial kernel scaffold; baseline (speedup 1.0000x reference)
import jax
import jax.numpy as jnp
from jax.experimental import pallas as pl


def kernel(x, edge_index, edge_attr, Wq, bq, Wk, bk, Wv, bv, Wo, bo, We, be):
    raise NotImplementedError("write your pallas kernel here")


# speedup vs baseline: 10.0449x; 10.0449x over previous
import functools
import math

import jax
import jax.numpy as jnp
import numpy as np
from jax import lax
from jax.experimental import pallas as pl
from jax.experimental.pallas import tpu as pltpu
from jax.experimental.pallas import tpu_sc as plsc

N = 10000
E = 320000
HIDDEN = 150
H = 5
HD = 30
S = 32
W = H * S
D_EDGE = 16
ESC_W = 8
SCALE = math.sqrt(HD)

NC = 2
NS = 16
NW = NC * NS
EPT = E // NW
B = 48
NCHUNK = EPT // B
BT = EPT - NCHUNK * B
GRP = B // 16
ROWS_PT = 624
ROWS_TAIL = N - NS * ROWS_PT

BN = 1000
EB = 20000


def _proj_body(x_ref, wq_ref, bq_ref, wk_ref, bk_ref, wv_ref, bv_ref,
               q_ref, k_ref, v_ref):
    xb = x_ref[...]
    q_ref[...] = jnp.dot(xb, wq_ref[...],
                         preferred_element_type=jnp.float32) + bq_ref[...]
    k_ref[...] = jnp.dot(xb, wk_ref[...],
                         preferred_element_type=jnp.float32) + bk_ref[...]
    v_ref[...] = jnp.dot(xb, wv_ref[...],
                         preferred_element_type=jnp.float32) + bv_ref[...]


_proj_call = pl.pallas_call(
    _proj_body,
    grid=(N // BN,),
    in_specs=[
        pl.BlockSpec((BN, HIDDEN), lambda i: (i, 0)),
        pl.BlockSpec((HIDDEN, W), lambda i: (0, 0)),
        pl.BlockSpec((1, W), lambda i: (0, 0)),
        pl.BlockSpec((HIDDEN, W), lambda i: (0, 0)),
        pl.BlockSpec((1, W), lambda i: (0, 0)),
        pl.BlockSpec((HIDDEN, W), lambda i: (0, 0)),
        pl.BlockSpec((1, W), lambda i: (0, 0)),
    ],
    out_specs=[
        pl.BlockSpec((BN, W), lambda i: (i, 0)),
        pl.BlockSpec((BN, W), lambda i: (i, 0)),
        pl.BlockSpec((BN, W), lambda i: (i, 0)),
    ],
    out_shape=[
        jax.ShapeDtypeStruct((N, W), jnp.float32),
        jax.ShapeDtypeStruct((N, W), jnp.float32),
        jax.ShapeDtypeStruct((N, W), jnp.float32),
    ],
)


def _esc_body(ea_ref, we_ref, be_ref, o_ref):
    o_ref[...] = jnp.dot(ea_ref[...], we_ref[...],
                         preferred_element_type=jnp.float32) + be_ref[...]


_esc_call = pl.pallas_call(
    _esc_body,
    grid=(E // EB,),
    in_specs=[
        pl.BlockSpec((EB, D_EDGE), lambda i: (i, 0)),
        pl.BlockSpec((D_EDGE, ESC_W), lambda i: (0, 0)),
        pl.BlockSpec((1, ESC_W), lambda i: (0, 0)),
    ],
    out_specs=pl.BlockSpec((EB, ESC_W), lambda i: (i, 0)),
    out_shape=jax.ShapeDtypeStruct((E, ESC_W), jnp.float32),
)


def _edge_group(qbuf, kbuf, vbuf, escv, e_idx, mask=None):
    escs = [plsc.load_gather(escv, [e_idx, jnp.full((16,), h, jnp.int32)],
                             mask=mask)
            for h in range(H)]
    for h in range(H):
        off = h * S
        acc = None
        for d in range(HD):
            cvec = jnp.full((16,), off + d, jnp.int32)
            qd = plsc.load_gather(qbuf, [e_idx, cvec], mask=mask)
            kd = plsc.load_gather(kbuf, [e_idx, cvec], mask=mask)
            t = qd * kd
            acc = t if acc is None else acc + t
        p = jnp.exp(acc + escs[h])
        for d in range(HD):
            cvec = jnp.full((16,), off + d, jnp.int32)
            vd = plsc.load_gather(vbuf, [e_idx, cvec], mask=mask)
            plsc.store_scatter(vbuf, [e_idx, cvec], p * vd, mask=mask)
        plsc.store_scatter(vbuf, [e_idx, jnp.full((16,), off + HD, jnp.int32)],
                           p, mask=mask)


def _sc_body(q_hbm, k_hbm, v_hbm, row_hbm, col_hbm, esc_hbm, zeros_hbm,
             out_hbm, rowv, colv, qbuf, kbuf, vbuf, escv, rowv_t, colv_t,
             aggs, sem_q, sem_k, sem_v):
    c = lax.axis_index("c")
    s = lax.axis_index("s")
    wid = c * NS + s
    ebase = wid * EPT

    pltpu.sync_copy(zeros_hbm, aggs.at[pl.ds(s * ROWS_PT, ROWS_PT)])

    @pl.when(s == NS - 1)
    def _zero_tail():
        pltpu.sync_copy(zeros_hbm.at[pl.ds(0, ROWS_TAIL)],
                        aggs.at[pl.ds(NS * ROWS_PT, ROWS_TAIL)])

    plsc.subcore_barrier()

    lane = lax.iota(jnp.int32, 16)

    def chunk_body(i, carry):
        eb = ebase + i * B
        pltpu.sync_copy(row_hbm.at[pl.ds(eb, B)], rowv)
        pltpu.sync_copy(col_hbm.at[pl.ds(eb, B)], colv)
        pltpu.sync_copy(esc_hbm.at[pl.ds(eb, B)], escv)
        cp_q = pltpu.async_copy(q_hbm.at[rowv], qbuf, sem_q)
        cp_k = pltpu.async_copy(k_hbm.at[colv], kbuf, sem_k)
        cp_v = pltpu.async_copy(v_hbm.at[colv], vbuf, sem_v)
        cp_q.wait()
        cp_k.wait()
        cp_v.wait()

        def group_body(g, carry2):
            _edge_group(qbuf, kbuf, vbuf, escv, lane + g * 16)
            return carry2

        lax.fori_loop(0, GRP, group_body, 0)
        pltpu.sync_copy(vbuf, aggs.at[rowv], add=True)
        return carry

    lax.fori_loop(0, NCHUNK, chunk_body, 0)

    tb = ebase + NCHUNK * B
    pltpu.sync_copy(row_hbm.at[pl.ds(tb, BT)], rowv_t)
    pltpu.sync_copy(col_hbm.at[pl.ds(tb, BT)], colv_t)
    pltpu.sync_copy(esc_hbm.at[pl.ds(tb, BT)], escv.at[pl.ds(0, BT)])
    pltpu.async_copy(q_hbm.at[rowv_t], qbuf.at[pl.ds(0, BT)], sem_q).wait()
    pltpu.async_copy(k_hbm.at[colv_t], kbuf.at[pl.ds(0, BT)], sem_k).wait()
    pltpu.async_copy(v_hbm.at[colv_t], vbuf.at[pl.ds(0, BT)], sem_v).wait()
    _edge_group(qbuf, kbuf, vbuf, escv, lane)
    pltpu.sync_copy(vbuf.at[pl.ds(0, BT)], aggs.at[rowv_t], add=True)

    plsc.subcore_barrier()
    pltpu.sync_copy(aggs.at[pl.ds(s * ROWS_PT, ROWS_PT)],
                    out_hbm.at[pl.ds(c * N + s * ROWS_PT, ROWS_PT)])

    @pl.when(s == NS - 1)
    def _write_tail():
        pltpu.sync_copy(aggs.at[pl.ds(NS * ROWS_PT, ROWS_TAIL)],
                        out_hbm.at[pl.ds(c * N + NS * ROWS_PT, ROWS_TAIL)])


_sc_call = functools.partial(
    pl.kernel,
    out_type=jax.ShapeDtypeStruct((NC * N, W), jnp.float32),
    mesh=plsc.VectorSubcoreMesh(core_axis_name="c", subcore_axis_name="s"),
    scratch_types=[
        pltpu.VMEM((B,), jnp.int32),
        pltpu.VMEM((B,), jnp.int32),
        pltpu.VMEM((B, W), jnp.float32),
        pltpu.VMEM((B, W), jnp.float32),
        pltpu.VMEM((B, W), jnp.float32),
        pltpu.VMEM((B, ESC_W), jnp.float32),
        pltpu.VMEM((BT,), jnp.int32),
        pltpu.VMEM((BT,), jnp.int32),
        pltpu.VMEM_SHARED((N, W), jnp.float32),
        pltpu.SemaphoreType.DMA,
        pltpu.SemaphoreType.DMA,
        pltpu.SemaphoreType.DMA,
    ],
    compiler_params=pltpu.CompilerParams(use_tc_tiling_on_sc=False,
                                         needs_layout_passes=False),
)(_sc_body)


def _comb_body(agg_ref, smat_ref, wop_ref, bo_ref, o_ref):
    a = agg_ref[0] + agg_ref[1]
    den = jnp.dot(a, smat_ref[...],
                  preferred_element_type=jnp.float32) + 1e-8
    npad = a / den
    o_ref[...] = jnp.dot(npad, wop_ref[...],
                         preferred_element_type=jnp.float32) + bo_ref[...]


_comb_call = pl.pallas_call(
    _comb_body,
    grid=(N // BN,),
    in_specs=[
        pl.BlockSpec((2, BN, W), lambda i: (0, i, 0)),
        pl.BlockSpec((W, W), lambda i: (0, 0)),
        pl.BlockSpec((W, HIDDEN), lambda i: (0, 0)),
        pl.BlockSpec((1, HIDDEN), lambda i: (0, 0)),
    ],
    out_specs=pl.BlockSpec((BN, HIDDEN), lambda i: (i, 0)),
    out_shape=jax.ShapeDtypeStruct((N, HIDDEN), jnp.float32),
)

_SMAT = np.zeros((W, W), np.float32)
for _h in range(H):
    _SMAT[S * _h + HD, S * _h:S * _h + S] = 1.0


def _pad_heads_cols(Wm, bm, scale):
    Wr = Wm.reshape(HIDDEN, H, HD) * scale
    Wp = jnp.concatenate(
        [Wr, jnp.zeros((HIDDEN, H, S - HD), Wm.dtype)], axis=2)
    br = bm.reshape(H, HD) * scale
    bp = jnp.concatenate([br, jnp.zeros((H, S - HD), bm.dtype)], axis=1)
    return Wp.reshape(HIDDEN, W), bp.reshape(1, W)


def kernel(x, edge_index, edge_attr, Wq, bq, Wk, bk, Wv, bv, Wo, bo, We, be):
    row = edge_index[0].astype(jnp.int32)
    col = edge_index[1].astype(jnp.int32)

    wq_p, bq_p = _pad_heads_cols(Wq, bq, 1.0 / SCALE)
    wk_p, bk_p = _pad_heads_cols(Wk, bk, 1.0)
    wv_p, bv_p = _pad_heads_cols(Wv, bv, 1.0)

    we_p = jnp.concatenate(
        [We, jnp.zeros((D_EDGE, ESC_W - H), We.dtype)], axis=1)
    be_p = jnp.concatenate(
        [be, jnp.zeros((ESC_W - H,), be.dtype)], axis=0).reshape(1, ESC_W)

    wo_p = jnp.concatenate(
        [Wo.reshape(H, HD, HIDDEN),
         jnp.zeros((H, S - HD, HIDDEN), Wo.dtype)], axis=1).reshape(W, HIDDEN)

    q_tab, k_tab, v_tab = _proj_call(x, wq_p, bq_p, wk_p, bk_p, wv_p, bv_p)
    esc = _esc_call(edge_attr, we_p, be_p)
    zeros = jnp.zeros((ROWS_PT, W), jnp.float32)
    agg = _sc_call(q_tab, k_tab, v_tab, row, col, esc, zeros)
    out = _comb_call(agg.reshape(NC, N, W), jnp.asarray(_SMAT), wo_p,
                     bo.reshape(1, HIDDEN))
    return out
